# baseline (speedup 1.0000x reference)
import functools

import jax
import jax.numpy as jnp
from jax import lax
from jax.experimental import pallas as pl
from jax.experimental.pallas import tpu as pltpu
from jax.experimental.pallas import tpu_sc as plsc

L = 16
CH = 8
NBUF = 4


@functools.partial(jax.jit, static_argnames=("num_tokens", "hidden", "num_experts", "capacity"))
def _dispatch(token_hidden, expert_idx, slot_idx, expert_offsets,
              num_tokens, hidden, num_experts, capacity):
    info = plsc.get_sparse_core_info()
    nc, ns = info.num_cores, info.num_subcores
    nw = nc * ns
    rows = num_experts * capacity
    cap_ch = capacity // CH

    mesh = plsc.VectorSubcoreMesh(core_axis_name="c", subcore_axis_name="s")

    @functools.partial(
        pl.kernel,
        out_type=jax.ShapeDtypeStruct((rows, hidden), token_hidden.dtype),
        mesh=mesh,
        compiler_params=pltpu.CompilerParams(needs_layout_passes=False),
        scratch_types=[
            pltpu.VMEM((num_tokens,), jnp.int32),
            pltpu.VMEM((num_tokens,), jnp.int32),
            pltpu.VMEM((num_experts + 1,), jnp.int32),
            pltpu.VMEM((capacity + L,), jnp.int32),
            pltpu.VMEM((NBUF, CH, hidden), token_hidden.dtype),
            pltpu.VMEM((L, hidden), token_hidden.dtype),
        ] + [pltpu.SemaphoreType.DMA] * (2 * NBUF + 2),
    )
    def k(th_hbm, e_hbm, s_hbm, off_hbm, out_hbm,
          e_v, s_v, off_v, tok_v, stage, zbuf, *sems):
        sem_g = sems[:NBUF]
        sem_w = sems[NBUF:2 * NBUF]
        sem_z = sems[2 * NBUF]
        sem_m = sems[2 * NBUF + 1]
        wid = lax.axis_index("s") * nc + lax.axis_index("c")
        my_e = wid // 2
        parity = wid % 2

        pltpu.sync_copy(e_hbm, e_v)
        pltpu.sync_copy(s_hbm, s_v)
        pltpu.sync_copy(off_hbm, off_v)

        e_splat = jnp.full((L,), my_e, jnp.int32)
        lane = lax.iota(jnp.int32, L)

        def scan_step(i, m):
            ev = e_v[pl.ds(i * L, L)]
            sv = s_v[pl.ds(i * L, L)]
            msk = ev == e_splat
            plsc.store_scatter(tok_v, [sv], i * L + lane, mask=msk)
            return jnp.maximum(m, jnp.where(msk, sv, -1))

        m = lax.fori_loop(0, num_tokens // L, scan_step,
                          jnp.full((L,), -1, jnp.int32))
        cnt = jnp.max(m) + 1

        lo_e = jnp.max(plsc.load_gather(off_v, [e_splat]))

        nfd = cnt // CH
        nd = jnp.maximum(0, (nfd - parity + 1) // 2)

        def out_row(k):
            c = parity + 2 * k
            return pl.multiple_of(lo_e + c * CH, 8)

        def body(k4, _):
            for u in range(NBUF):
                kk = k4 * NBUF + u

                @pl.when(jnp.logical_and(kk >= NBUF, kk < nd))
                def _(u=u, kk=kk):
                    pltpu.make_async_copy(
                        stage.at[u], out_hbm.at[pl.ds(0, CH)], sem_w[u]).wait()

                @pl.when(kk < nd)
                def _(u=u, kk=kk):
                    tl_ref = tok_v.at[pl.ds(pl.multiple_of(
                        (parity + 2 * kk) * CH, 8), CH)]
                    pltpu.async_copy(th_hbm.at[tl_ref], stage.at[u], sem_g[u])

                up = (u - 1) % NBUF

                @pl.when(jnp.logical_and(kk >= 1, kk - 1 < nd))
                def _(u=u, up=up, kk=kk):
                    pltpu.make_async_copy(
                        th_hbm.at[pl.ds(0, CH)], stage.at[up], sem_g[up]).wait()
                    pltpu.async_copy(stage.at[up],
                                     out_hbm.at[pl.ds(out_row(kk - 1), CH)],
                                     sem_w[up])
            return 0

        lax.fori_loop(0, nd // NBUF + 1, body, 0)

        for u in range(NBUF):
            @pl.when(nd > u)
            def _(u=u):
                pltpu.make_async_copy(
                    stage.at[u], out_hbm.at[pl.ds(0, CH)], sem_w[u]).wait()

        cm = cnt // CH
        has_mixed = jnp.logical_and(cnt % CH != 0, cm % 2 == parity)

        @pl.when(has_mixed)
        def _():
            tl = tok_v[pl.ds(pl.multiple_of(cm * CH, 8), L)]
            slot_g = cm * CH + lane
            ivec = jnp.where(slot_g < cnt, tl, 0)
            pltpu.async_copy(th_hbm.at[ivec], zbuf, sem_m).wait()

            zeros16 = jnp.zeros((L,), token_hidden.dtype)

            def fix_row(r, _):
                @pl.when(cm * CH + r >= cnt)
                def _():
                    def fix_col(cc, _2):
                        zbuf[r, pl.ds(cc * L, L)] = zeros16
                        return 0
                    lax.fori_loop(0, hidden // L, fix_col, 0)
                return 0

            lax.fori_loop(0, CH, fix_row, 0)
            pltpu.sync_copy(
                zbuf.at[pl.ds(0, CH)],
                out_hbm.at[pl.ds(pl.multiple_of(lo_e + cm * CH, 8), CH)])

        zeros16 = jnp.zeros((L,), token_hidden.dtype)

        def mz_row(i, _):
            def mz_col(cc, _2):
                zbuf[i, pl.ds(cc * L, L)] = zeros16
                return 0
            lax.fori_loop(0, hidden // L, mz_col, 0)
            return 0

        lax.fori_loop(0, L, mz_row, 0)

        cz0 = (cnt + CH - 1) // CH
        czs = cz0 + ((cz0 + parity) % 2)
        nz = jnp.maximum(0, (cap_ch - czs + 1) // 2)

        def zero_chunk(q, _):
            c = czs + 2 * q
            pltpu.async_copy(
                zbuf.at[pl.ds(0, CH)],
                out_hbm.at[pl.ds(pl.multiple_of(lo_e + c * CH, 8), CH)], sem_z)

            @pl.when(q > 0)
            def _():
                pltpu.make_async_copy(
                    zbuf.at[pl.ds(0, CH)], out_hbm.at[pl.ds(0, CH)],
                    sem_z).wait()
            return 0

        lax.fori_loop(0, nz, zero_chunk, 0)

        @pl.when(nz > 0)
        def _():
            pltpu.make_async_copy(
                zbuf.at[pl.ds(0, CH)], out_hbm.at[pl.ds(0, CH)], sem_z).wait()

    return k(token_hidden, expert_idx, slot_idx, expert_offsets)


def kernel(token_hidden, expert_idx, slot_idx, expert_offsets):
    num_tokens, hidden = token_hidden.shape
    num_experts = expert_offsets.shape[0] - 1
    return _dispatch(token_hidden, expert_idx, slot_idx, expert_offsets,
                     num_tokens=num_tokens, hidden=hidden,
                     num_experts=num_experts, capacity=1024)


# speedup vs baseline: 1.1086x; 1.1086x over previous
import functools

import jax
import jax.numpy as jnp
from jax import lax
from jax.experimental import pallas as pl
from jax.experimental.pallas import tpu as pltpu
from jax.experimental.pallas import tpu_sc as plsc

L = 16
CH = 8
NBUF = 4


@functools.partial(jax.jit, static_argnames=("num_tokens", "hidden", "num_experts", "capacity"))
def _dispatch(token_hidden, expert_idx, slot_idx, expert_offsets,
              num_tokens, hidden, num_experts, capacity):
    info = plsc.get_sparse_core_info()
    nc, ns = info.num_cores, info.num_subcores
    nw = nc * ns
    rows = num_experts * capacity
    n_chunks = rows // CH
    cpw = n_chunks // nw

    mesh = plsc.VectorSubcoreMesh(core_axis_name="c", subcore_axis_name="s")

    @functools.partial(
        pl.kernel,
        out_type=jax.ShapeDtypeStruct((rows, hidden), token_hidden.dtype),
        mesh=mesh,
        compiler_params=pltpu.CompilerParams(needs_layout_passes=False),
        scratch_types=[
            pltpu.VMEM((num_tokens,), jnp.int32),
            pltpu.VMEM((num_tokens,), jnp.int32),
            pltpu.VMEM((num_experts + 1,), jnp.int32),
            pltpu.VMEM((rows + L,), jnp.int32),
            pltpu.VMEM((NBUF * L,), jnp.int32),
            pltpu.VMEM((NBUF, CH, hidden), token_hidden.dtype),
            pltpu.VMEM((CH, hidden), token_hidden.dtype),
        ] + [pltpu.SemaphoreType.DMA] * (2 * NBUF),
    )
    def k(th_hbm, e_hbm, s_hbm, off_hbm, out_hbm,
          e_v, s_v, off_v, inv_v, idx_v, stage, zbuf, *sems):
        sem_g = sems[:NBUF]
        sem_w = sems[NBUF:]
        wid = lax.axis_index("s") * nc + lax.axis_index("c")
        lane = lax.iota(jnp.int32, L)
        zeros16 = jnp.zeros((L,), token_hidden.dtype)
        neg1 = jnp.full((L,), -1, jnp.int32)

        pltpu.sync_copy(e_hbm, e_v)
        pltpu.sync_copy(s_hbm, s_v)
        pltpu.sync_copy(off_hbm, off_v)

        def mz_row(i, _):
            def mz_col(cc, _2):
                zbuf[i, pl.ds(cc * L, L)] = zeros16
                return 0
            lax.fori_loop(0, hidden // L, mz_col, 0)
            return 0

        lax.fori_loop(0, CH, mz_row, 0)

        def init_inv(i, _):
            inv_v[pl.ds(i * L, L)] = neg1
            return 0

        lax.fori_loop(0, (rows + L) // L, init_inv, 0)

        def scan_step(i, _):
            ev = e_v[pl.ds(i * L, L)]
            sv = s_v[pl.ds(i * L, L)]
            row = plsc.load_gather(off_v, [ev]) + sv
            plsc.store_scatter(inv_v, [row], i * L + lane)
            return 0

        lax.fori_loop(0, num_tokens // L, scan_step, 0)

        def chunk_row(kk):
            return pl.multiple_of((wid + nw * kk) * CH, 8)

        def classify(kk):
            w = inv_v[pl.ds(chunk_row(kk), L)]
            first8 = lane < CH
            any_data = jnp.max(jnp.where(jnp.logical_and(first8, w >= 0),
                                         1, 0)) > 0
            any_sent = jnp.min(jnp.where(first8, w, 0)) < 0
            return w, any_data, jnp.logical_and(any_data, any_sent)

        def fire_write_prev(kk, up):
            w, any_data, mixed = classify(kk)

            @pl.when(any_data)
            def _():
                pltpu.make_async_copy(
                    th_hbm.at[pl.ds(0, CH)], stage.at[up], sem_g[up]).wait()

                @pl.when(mixed)
                def _():
                    for r in range(CH):
                        s_r = jnp.min(jnp.where(lane == r, w, 0))

                        @pl.when(s_r < 0)
                        def _(r=r):
                            def fix_col(cc, _2):
                                stage[up, r, pl.ds(cc * L, L)] = zeros16
                                return 0
                            lax.fori_loop(0, hidden // L, fix_col, 0)

                pltpu.async_copy(stage.at[up],
                                 out_hbm.at[pl.ds(chunk_row(kk), CH)],
                                 sem_w[up])

            @pl.when(jnp.logical_not(any_data))
            def _():
                pltpu.async_copy(zbuf,
                                 out_hbm.at[pl.ds(chunk_row(kk), CH)],
                                 sem_w[up])

        def body(k4, _):
            for u in range(NBUF):
                kk = k4 * NBUF + u

                @pl.when(kk >= NBUF)
                def _(u=u):
                    pltpu.make_async_copy(
                        stage.at[u], out_hbm.at[pl.ds(0, CH)], sem_w[u]).wait()

                w, any_data, _mx = classify(kk)

                @pl.when(any_data)
                def _(u=u, w=w):
                    idx_v[pl.ds(u * L, L)] = jnp.maximum(w, 0)
                    pltpu.async_copy(
                        th_hbm.at[idx_v.at[pl.ds(u * L, CH)]],
                        stage.at[u], sem_g[u])

                @pl.when(kk >= 1)
                def _(u=u, kk=kk):
                    fire_write_prev(kk - 1, (u - 1) % NBUF)
            return 0

        lax.fori_loop(0, cpw // NBUF, body, 0)

        fire_write_prev(cpw - 1, (cpw - 1) % NBUF)
        for u in range(NBUF):
            pltpu.make_async_copy(
                stage.at[u], out_hbm.at[pl.ds(0, CH)], sem_w[u]).wait()

    return k(token_hidden, expert_idx, slot_idx, expert_offsets)


def kernel(token_hidden, expert_idx, slot_idx, expert_offsets):
    num_tokens, hidden = token_hidden.shape
    num_experts = expert_offsets.shape[0] - 1
    return _dispatch(token_hidden, expert_idx, slot_idx, expert_offsets,
                     num_tokens=num_tokens, hidden=hidden,
                     num_experts=num_experts, capacity=1024)
